# baseline (speedup 1.0000x reference)
import functools

import jax
import jax.numpy as jnp
from jax import lax
from jax.experimental import pallas as pl
from jax.experimental.pallas import tpu as pltpu
from jax.experimental.pallas import tpu_sc as plsc

N_ROWS = 16384
D_OBS = 128
N_PH = 8
D_OUT = D_OBS + N_PH
NC = 2
NS = 16
NW = NC * NS
ROWS_PER_W = N_ROWS // NW
CHUNK = 128
N_CHUNKS = ROWS_PER_W // CHUNK


def _sc_body(obs_hbm, ph_hbm, out_hbm, obs_v, ph_v, t_v, sem):
    wid = lax.axis_index("s") * NC + lax.axis_index("c")
    base = wid * ROWS_PER_W

    lanes = lax.broadcasted_iota(jnp.int32, (16,), 0)

    for k in range(N_CHUNKS):
        r0 = base + k * CHUNK
        cp = pltpu.make_async_copy(
            obs_hbm.at[pl.ds(r0, CHUNK)], obs_v.at[:, pl.ds(0, D_OBS)], sem
        )
        cp.start()
        pltpu.sync_copy(ph_hbm.at[pl.ds(r0, CHUNK)], ph_v)

        def oh_body(g, carry):
            ph = ph_v[pl.ds(g * 16, 16)]
            for c in range(N_PH):
                t_v[D_OBS + c, pl.ds(g * 16, 16)] = jnp.where(
                    ph == c, 1.0, 0.0
                ).astype(jnp.float32)
            return carry

        lax.fori_loop(0, CHUNK // 16, oh_body, 0, unroll=2)

        cp.wait()

        def tr_body(g, carry):
            row_idx = g * 16 + lanes
            for c in range(D_OBS):
                t_v[c, pl.ds(g * 16, 16)] = plsc.load_gather(
                    obs_v, [row_idx, jnp.full((16,), c, jnp.int32)]
                )
            return carry

        lax.fori_loop(0, CHUNK // 16, tr_body, 0)

        pltpu.sync_copy(t_v, out_hbm.at[:, pl.ds(r0, CHUNK)])


_mesh = plsc.VectorSubcoreMesh(core_axis_name="c", subcore_axis_name="s")

_sc_call = functools.partial(
    pl.kernel,
    mesh=_mesh,
    out_type=jax.ShapeDtypeStruct((D_OUT, N_ROWS), jnp.float32),
    scratch_types=[
        pltpu.VMEM((CHUNK, D_OUT), jnp.float32),
        pltpu.VMEM((CHUNK,), jnp.int32),
        pltpu.VMEM((D_OUT, CHUNK), jnp.float32),
        pltpu.SemaphoreType.DMA,
    ],
    compiler_params=pltpu.CompilerParams(needs_layout_passes=False),
)(_sc_body)


def kernel(obs, phases):
    t = _sc_call(obs, phases.astype(jnp.int32))
    return t.T


# speedup vs baseline: 3.5041x; 3.5041x over previous
import functools

import jax
import jax.numpy as jnp
from jax import lax
from jax.experimental import pallas as pl
from jax.experimental.pallas import tpu as pltpu
from jax.experimental.pallas import tpu_sc as plsc

N_ROWS = 16384
D_OBS = 128
N_PH = 8
D_OUT = D_OBS + N_PH
NC = 2
NS = 16
NW = NC * NS
ROWS_PER_W = N_ROWS // NW
CHUNK = 128
N_CHUNKS = ROWS_PER_W // CHUNK
GROUPS = CHUNK // 16


def _sc_body(obs_hbm, ph_hbm, out_hbm, obs_v, ph_v, t_v, sem_in, sem_out):
    wid = lax.axis_index("s") * NC + lax.axis_index("c")
    base = wid * ROWS_PER_W

    lanes = lax.broadcasted_iota(jnp.int32, (16,), 0)

    def in_copy(k):
        return pltpu.make_async_copy(
            obs_hbm.at[pl.ds(base + k * CHUNK, CHUNK)],
            obs_v.at[k % 2, :, pl.ds(0, D_OBS)],
            sem_in,
        )

    def out_copy(k):
        return pltpu.make_async_copy(
            t_v.at[k % 2],
            out_hbm.at[:, pl.ds(base + k * CHUNK, CHUNK)],
            sem_out,
        )

    in_copy(0).start()
    for k in range(N_CHUNKS):
        b = k % 2
        if k + 1 < N_CHUNKS:
            in_copy(k + 1).start()
        pltpu.sync_copy(ph_hbm.at[pl.ds(base + k * CHUNK, CHUNK)], ph_v)
        ob = obs_v.at[b]
        tb = t_v.at[b]

        if k >= 2:
            out_copy(k - 2).wait()
        in_copy(k).wait()

        @functools.partial(plsc.parallel_loop, 0, GROUPS, unroll=2)
        def _oh(g):
            ph = ph_v[pl.ds(g * 16, 16)]
            for c in range(N_PH):
                tb[D_OBS + c, pl.ds(g * 16, 16)] = jnp.where(
                    ph == c, 1.0, 0.0
                ).astype(jnp.float32)

        @functools.partial(plsc.parallel_loop, 0, GROUPS * D_OBS, unroll=8)
        def _tr(i):
            g = i // D_OBS
            c = i % D_OBS
            tb[c, pl.ds(g * 16, 16)] = plsc.load_gather(
                ob, [g * 16 + lanes, jnp.full((16,), c, jnp.int32)]
            )

        out_copy(k).start()

    out_copy(N_CHUNKS - 2).wait()
    out_copy(N_CHUNKS - 1).wait()


_mesh = plsc.VectorSubcoreMesh(core_axis_name="c", subcore_axis_name="s")

_sc_call = functools.partial(
    pl.kernel,
    mesh=_mesh,
    out_type=jax.ShapeDtypeStruct((D_OUT, N_ROWS), jnp.float32),
    scratch_types=[
        pltpu.VMEM((2, CHUNK, D_OUT), jnp.float32),
        pltpu.VMEM((CHUNK,), jnp.int32),
        pltpu.VMEM((2, D_OUT, CHUNK), jnp.float32),
        pltpu.SemaphoreType.DMA,
        pltpu.SemaphoreType.DMA,
    ],
    compiler_params=pltpu.CompilerParams(needs_layout_passes=False),
)(_sc_body)


def kernel(obs, phases):
    t = _sc_call(obs, phases.astype(jnp.int32))
    return t.T


# speedup vs baseline: 3.5106x; 1.0018x over previous
import functools

import jax
import jax.numpy as jnp
from jax import lax
from jax.experimental import pallas as pl
from jax.experimental.pallas import tpu as pltpu
from jax.experimental.pallas import tpu_sc as plsc

N_ROWS = 16384
D_OBS = 128
N_PH = 8
D_OUT = D_OBS + N_PH
NC = 2
NS = 16
NW = NC * NS
ROWS_PER_W = N_ROWS // NW
CHUNK = 128
N_CHUNKS = ROWS_PER_W // CHUNK
GROUPS = CHUNK // 16


def _sc_body(obs_hbm, ph_hbm, out_hbm, obs_v, ph_v, t_v, sem_in, sem_out):
    wid = lax.axis_index("s") * NC + lax.axis_index("c")
    base = wid * ROWS_PER_W

    lanes = lax.broadcasted_iota(jnp.int32, (16,), 0)

    def in_copy(k):
        return pltpu.make_async_copy(
            obs_hbm.at[pl.ds(base + k * CHUNK, CHUNK)],
            obs_v.at[k % 2, :, pl.ds(0, D_OBS)],
            sem_in,
        )

    def out_copy(k):
        return pltpu.make_async_copy(
            t_v.at[k % 2],
            out_hbm.at[:, pl.ds(base + k * CHUNK, CHUNK)],
            sem_out,
        )

    in_copy(0).start()
    pltpu.sync_copy(ph_hbm.at[pl.ds(base, ROWS_PER_W)], ph_v)
    for k in range(N_CHUNKS):
        b = k % 2
        if k + 1 < N_CHUNKS:
            in_copy(k + 1).start()
        ob = obs_v.at[b]
        tb = t_v.at[b]

        if k >= 2:
            out_copy(k - 2).wait()
        in_copy(k).wait()

        @functools.partial(plsc.parallel_loop, 0, GROUPS, unroll=2)
        def _oh(g):
            ph = ph_v[pl.ds(k * CHUNK + g * 16, 16)]
            for c in range(N_PH):
                tb[D_OBS + c, pl.ds(g * 16, 16)] = jnp.where(
                    ph == c, 1.0, 0.0
                ).astype(jnp.float32)

        @functools.partial(plsc.parallel_loop, 0, GROUPS * D_OBS, unroll=16)
        def _tr(i):
            g = i // D_OBS
            c = i % D_OBS
            tb[c, pl.ds(g * 16, 16)] = plsc.load_gather(
                ob, [g * 16 + lanes, jnp.full((16,), c, jnp.int32)]
            )

        out_copy(k).start()

    out_copy(N_CHUNKS - 2).wait()
    out_copy(N_CHUNKS - 1).wait()


_mesh = plsc.VectorSubcoreMesh(core_axis_name="c", subcore_axis_name="s")

_sc_call = functools.partial(
    pl.kernel,
    mesh=_mesh,
    out_type=jax.ShapeDtypeStruct((D_OUT, N_ROWS), jnp.float32),
    scratch_types=[
        pltpu.VMEM((2, CHUNK, D_OUT), jnp.float32),
        pltpu.VMEM((ROWS_PER_W,), jnp.int32),
        pltpu.VMEM((2, D_OUT, CHUNK), jnp.float32),
        pltpu.SemaphoreType.DMA,
        pltpu.SemaphoreType.DMA,
    ],
    compiler_params=pltpu.CompilerParams(needs_layout_passes=False),
)(_sc_body)


def kernel(obs, phases):
    t = _sc_call(obs, phases.astype(jnp.int32))
    return t.T
